# Initial kernel scaffold; baseline (speedup 1.0000x reference)
import jax
import jax.numpy as jnp
from jax.experimental import pallas as pl


def kernel(x, edge_index, edge_attr, params):
    raise NotImplementedError("write your pallas kernel here")


# speedup vs baseline: 4.9680x; 4.9680x over previous
import functools

import jax
import jax.numpy as jnp
from jax import lax
from jax.experimental import pallas as pl
from jax.experimental.pallas import tpu as pltpu
from jax.experimental.pallas import tpu_sc as plsc

NN = 10000
NP = 10240
EE = 160000
FF = 128
HH = 128
BLK = 640
NBLK = NP // BLK

NSUB = 16
CW = 128
CPW = 80
EP = NSUB * CPW * CW
RPT = NP // NSUB

_P = jax.lax.Precision.HIGHEST


def _sc_agg_build():
    mesh = plsc.VectorSubcoreMesh(core_axis_name="c", subcore_axis_name="s",
                                  num_cores=1)
    out_type = [jax.ShapeDtypeStruct((NP, HH), jnp.float32)]
    scratch = [
        pltpu.VMEM((CPW, CW), jnp.int32),
        pltpu.VMEM((CPW, CW), jnp.int32),
        pltpu.VMEM((CPW, CW), jnp.float32),
        pltpu.VMEM((CW, HH), jnp.float32),
        pltpu.VMEM((8, HH), jnp.float32),
        pltpu.VMEM_SHARED((NP, HH), jnp.float32),
        pltpu.SemaphoreType.DMA,
    ]

    def body(tab, src3, dst3, ew3, outp,
             src_v, dst_v, ew_v, rows_v, zbuf, acc, sem):
        s = lax.axis_index("s")
        rbase = pl.multiple_of(s * RPT, 8)

        pltpu.sync_copy(src3.at[s], src_v)
        pltpu.sync_copy(dst3.at[s], dst_v)
        pltpu.sync_copy(ew3.at[s], ew_v)

        zero16 = jnp.zeros((16,), jnp.float32)

        def fill(i, cc):
            for kk in range(HH // 16):
                zbuf[i, pl.ds(kk * 16, 16)] = zero16
            return cc
        lax.fori_loop(0, 8, fill, 0)

        @pl.loop(0, RPT // 8)
        def zero(repi):
            zoff = pl.multiple_of(rbase + repi * 8, 8)
            pltpu.sync_copy(zbuf, acc.at[pl.ds(zoff, 8)])
        plsc.subcore_barrier()

        @pl.loop(0, CPW)
        def chunk(g):
            pltpu.async_copy(tab.at[src_v.at[g]], rows_v, sem).wait()

            def scale(j, c2):
                ew16 = ew_v[g, pl.ds(j * 16, 16)]
                for l in range(16):
                    ew_s = ew16[l]
                    e = j * 16 + l
                    for kk in range(HH // 16):
                        sl = pl.ds(kk * 16, 16)
                        rows_v[e, sl] = rows_v[e, sl] * ew_s
                return c2
            lax.fori_loop(0, CW // 16, scale, 0)

            pltpu.sync_copy(rows_v, acc.at[dst_v.at[g]], add=True)

        plsc.subcore_barrier()
        pltpu.sync_copy(acc.at[pl.ds(rbase, RPT)],
                        outp.at[pl.ds(rbase, RPT)])

    return pl.kernel(body, out_type=out_type, mesh=mesh,
                     scratch_types=scratch)


_sc_agg = _sc_agg_build()


def _chained_agg(tab, src3, dst3, ew3, token):
    if token is not None:
        tab, _ = lax.optimization_barrier((tab, token))
    (outp,) = _sc_agg(tab, src3, dst3, ew3)
    return outp


def _enc_body(x_ref, p_ref, cnt_ref, wt_ref, wb_ref, b_ref, o_ref):
    rinv = 1.0 / jnp.maximum(cnt_ref[:, 0:1], 1.0)
    agg = p_ref[...] * rinv
    o_ref[...] = (
        jnp.dot(x_ref[...], wt_ref[...], precision=_P)
        + jnp.dot(agg, wb_ref[...], precision=_P)
        + b_ref[...]
    )


def _encoder(xt, partial, counts, w, b):
    blk = lambda i: (i, 0)
    return pl.pallas_call(
        _enc_body,
        grid=(NBLK,),
        in_specs=[
            pl.BlockSpec((BLK, FF), blk),
            pl.BlockSpec((BLK, FF), blk),
            pl.BlockSpec((BLK, HH), blk),
            pl.BlockSpec((FF, HH), lambda i: (0, 0)),
            pl.BlockSpec((FF, HH), lambda i: (0, 0)),
            pl.BlockSpec((1, HH), lambda i: (0, 0)),
        ],
        out_specs=pl.BlockSpec((BLK, HH), blk),
        out_shape=jax.ShapeDtypeStruct((NP, HH), jnp.float32),
    )(xt, partial, counts, w[:FF], w[FF:], b.reshape(1, HH))


def _gate_body_t0(xs_ref, p_ref, cnt_ref, wxt_ref, wxb_ref, b_ref,
                  wo_ref, bo_ref, h_out, c_out, y_out):
    rinv = 1.0 / jnp.maximum(cnt_ref[:, 0:1], 1.0)
    aggx = p_ref[...] * rinv
    z = (
        jnp.dot(xs_ref[...], wxt_ref[...], precision=_P)
        + jnp.dot(aggx, wxb_ref[...], precision=_P)
        + b_ref[...]
    )
    i = jax.nn.sigmoid(z[:, 0 * HH:1 * HH])
    o = jax.nn.sigmoid(z[:, 2 * HH:3 * HH])
    g = jnp.tanh(z[:, 3 * HH:4 * HH])
    c = i * g
    h = o * jnp.tanh(c)
    c_out[...] = c
    h_out[...] = h
    y_out[...] = jnp.dot(h, wo_ref[...], precision=_P) + bo_ref[...]


def _gate_body(xs_ref, p_ref, h_ref, q_ref, cnt_ref, cst_ref,
               wxt_ref, wxb_ref, wht_ref, whb_ref, b_ref, wo_ref, bo_ref,
               h_out, c_out, y_out):
    rinv = 1.0 / jnp.maximum(cnt_ref[:, 0:1], 1.0)
    aggx = p_ref[...] * rinv
    aggh = q_ref[...] * rinv
    z = (
        jnp.dot(xs_ref[...], wxt_ref[...], precision=_P)
        + jnp.dot(aggx, wxb_ref[...], precision=_P)
        + jnp.dot(h_ref[...], wht_ref[...], precision=_P)
        + jnp.dot(aggh, whb_ref[...], precision=_P)
        + b_ref[...]
    )
    i = jax.nn.sigmoid(z[:, 0 * HH:1 * HH])
    f = jax.nn.sigmoid(z[:, 1 * HH:2 * HH])
    o = jax.nn.sigmoid(z[:, 2 * HH:3 * HH])
    g = jnp.tanh(z[:, 3 * HH:4 * HH])
    c = f * cst_ref[...] + i * g
    h = o * jnp.tanh(c)
    c_out[...] = c
    h_out[...] = h
    y_out[...] = jnp.dot(h, wo_ref[...], precision=_P) + bo_ref[...]


def _lstm_step(t, xs_t, aggxs_p, counts, h, aggh_p, cstate,
               wxt, wxb, wht, whb, bcat, wo, bo):
    blk = lambda i: (i, 0)
    fix = lambda i: (0, 0)
    wspec = pl.BlockSpec((HH, 4 * HH), fix)
    hb = pl.BlockSpec((BLK, HH), blk)
    out_specs = [
        pl.BlockSpec((BLK, HH), blk),
        pl.BlockSpec((BLK, HH), blk),
        pl.BlockSpec((BLK, 1), blk),
    ]
    out_shape = [
        jax.ShapeDtypeStruct((NP, HH), jnp.float32),
        jax.ShapeDtypeStruct((NP, HH), jnp.float32),
        jax.ShapeDtypeStruct((NP, 1), jnp.float32),
    ]
    if t == 0:
        return pl.pallas_call(
            _gate_body_t0,
            grid=(NBLK,),
            in_specs=[
                hb, hb, hb,
                wspec, wspec, pl.BlockSpec((1, 4 * HH), fix),
                pl.BlockSpec((HH, 1), fix), pl.BlockSpec((1, 1), fix),
            ],
            out_specs=out_specs,
            out_shape=out_shape,
        )(xs_t, aggxs_p, counts, wxt, wxb, bcat, wo, bo)
    return pl.pallas_call(
        _gate_body,
        grid=(NBLK,),
        in_specs=[
            hb, hb, hb, hb, hb, hb,
            wspec, wspec, wspec, wspec, pl.BlockSpec((1, 4 * HH), fix),
            pl.BlockSpec((HH, 1), fix), pl.BlockSpec((1, 1), fix),
        ],
        out_specs=out_specs,
        out_shape=out_shape,
    )(xs_t, aggxs_p, h, aggh_p, counts, cstate,
      wxt, wxb, wht, whb, bcat, wo, bo)


def kernel(x, edge_index, edge_attr, params):
    T = x.shape[0]
    npad = EP - EE
    src3 = jnp.pad(edge_index[0], (0, npad)).reshape(NSUB, CPW, CW)
    dst3 = jnp.pad(edge_index[1], (0, npad),
                   constant_values=NP - 1).reshape(NSUB, CPW, CW)
    ew3 = jnp.pad(edge_attr.reshape(EE), (0, npad)).reshape(NSUB, CPW, CW)

    xp = jnp.pad(x, ((0, 0), (0, NP - NN), (0, 0)))

    def stack(keys):
        ws = [params[k] for k in keys]
        top = jnp.concatenate([w[:HH] for w in ws], axis=1)
        bot = jnp.concatenate([w[HH:] for w in ws], axis=1)
        return top, bot

    gk = ('i', 'f', 'o', 'g')
    wxt, wxb = stack([f'W_x{k}' for k in gk])
    wht, whb = stack([f'W_h{k}' for k in gk])
    bcat = jnp.concatenate(
        [params[f'b_x{k}'] + params[f'b_h{k}'] for k in gk]).reshape(1, 4 * HH)
    wo = params['W_out']
    bo = params['b_out'].reshape(1, 1)

    counts = _chained_agg(jnp.ones((NP, FF), jnp.float32), src3, dst3,
                          jnp.ones((NSUB, CPW, CW), jnp.float32), None)
    tok = counts

    xs = []
    for t in range(T):
        p = _chained_agg(xp[t], src3, dst3, ew3, tok)
        tok = p
        xs.append(_encoder(xp[t], p, counts,
                           params['W_sage'], params['b_sage']))

    aggxs = []
    for xs_t in xs:
        p = _chained_agg(xs_t, src3, dst3, ew3, tok)
        tok = p
        aggxs.append(p)

    h = None
    cstate = None
    hs, cs = [], []
    y = None
    for t in range(T):
        aggh_p = None
        if t > 0:
            aggh_p = _chained_agg(h, src3, dst3, ew3, tok)
            tok = aggh_p
        h, cstate, y = _lstm_step(t, xs[t], aggxs[t], counts, h, aggh_p,
                                  cstate, wxt, wxb, wht, whb, bcat, wo, bo)
        hs.append(h[:NN])
        cs.append(cstate[:NN])

    hseq = jnp.stack(hs, axis=0)
    cseq = jnp.stack(cs, axis=0)
    out = y[:NN].reshape(1, NN, 1)
    return (out, hseq, cseq)


# speedup vs baseline: 6.4679x; 1.3019x over previous
import functools

import jax
import jax.numpy as jnp
from jax import lax
from jax.experimental import pallas as pl
from jax.experimental.pallas import tpu as pltpu
from jax.experimental.pallas import tpu_sc as plsc

NN = 10000
NP = 10240
EE = 160000
FF = 128
HH = 128
BLK = 640
NBLK = NP // BLK

NSUB = 16
CW = 128
CPW = 80
EP = NSUB * CPW * CW
RPT = NP // NSUB

_P = jax.lax.Precision.HIGHEST


ZR = 64


def _sc_agg_build():
    mesh = plsc.VectorSubcoreMesh(core_axis_name="c", subcore_axis_name="s",
                                  num_cores=1)
    out_type = [jax.ShapeDtypeStruct((NP, HH), jnp.float32)]
    scratch = [
        pltpu.VMEM((2, CW), jnp.int32),
        pltpu.VMEM((2, CW), jnp.int32),
        pltpu.VMEM((2, CW), jnp.int32),
        pltpu.VMEM((2, CW), jnp.int32),
        pltpu.VMEM((CW,), jnp.float32),
        pltpu.VMEM((CW,), jnp.float32),
        pltpu.VMEM((CW,), jnp.float32),
        pltpu.VMEM((CW,), jnp.float32),
        pltpu.VMEM((CW, HH), jnp.float32),
        pltpu.VMEM((CW, HH), jnp.float32),
        pltpu.VMEM((ZR, HH), jnp.float32),
        pltpu.VMEM_SHARED((NP, HH), jnp.float32),
        pltpu.SemaphoreType.DMA,
        pltpu.SemaphoreType.DMA,
        pltpu.SemaphoreType.DMA,
        pltpu.SemaphoreType.DMA,
        pltpu.SemaphoreType.DMA,
        pltpu.SemaphoreType.DMA,
        pltpu.SemaphoreType.DMA,
        pltpu.SemaphoreType.DMA,
    ]

    def body(tab, e4, ew4, outp,
             ebuf0, ebuf1, ebuf2, ebuf3, ewbuf0, ewbuf1, ewbuf2, ewbuf3,
             rows0, rows1, zbuf, acc,
             sem_e0, sem_e1, sem_e2, sem_e3, sem_g0, sem_g1,
             sem_s0, sem_s1):
        s = lax.axis_index("s")
        rbase = pl.multiple_of(s * RPT, 8)
        ebufs = (ebuf0, ebuf1, ebuf2, ebuf3)
        ewbufs = (ewbuf0, ewbuf1, ewbuf2, ewbuf3)
        rows = (rows0, rows1)
        sem_e = (sem_e0, sem_e1, sem_e2, sem_e3)
        sem_g = (sem_g0, sem_g1)
        sem_s = (sem_s0, sem_s1)
        me = e4.at[s]
        mw = ew4.at[s]

        zero16 = jnp.zeros((16,), jnp.float32)

        def fill(i, cc):
            for kk in range(HH // 16):
                zbuf[i, pl.ds(kk * 16, 16)] = zero16
            return cc
        lax.fori_loop(0, ZR, fill, 0)

        @pl.loop(0, RPT // ZR)
        def zero(repi):
            zoff = pl.multiple_of(rbase + repi * ZR, 8)
            pltpu.sync_copy(zbuf, acc.at[pl.ds(zoff, ZR)])
        plsc.subcore_barrier()

        pltpu.sync_copy(me.at[0], ebuf0)
        pltpu.sync_copy(mw.at[0], ewbuf0)
        pltpu.async_copy(tab.at[ebuf0.at[0]], rows0, sem_g0)
        pltpu.async_copy(me.at[1], ebuf1, sem_e1)
        pltpu.async_copy(mw.at[1], ewbuf1, sem_e1)

        def _scale(ewb, rb):
            def scale(j, c2):
                ew16 = ewb[pl.ds(j * 16, 16)]
                for l in range(16):
                    ew_s = ew16[l]
                    e = j * 16 + l
                    for kk in range(HH // 16):
                        sl = pl.ds(kk * 16, 16)
                        rb[e, sl] = rb[e, sl] * ew_s
                return c2
            lax.fori_loop(0, CW // 16, scale, 0)

        @pl.loop(0, CPW, step=4)
        def iters(g0):
            for q in range(4):
                g = g0 + q
                b = q % 2
                eb, rb = ebufs[q], rows[b]
                ebn, rbn = ebufs[(q + 1) % 4], rows[1 - b]
                ewbn = ewbufs[(q + 1) % 4]
                pltpu.make_async_copy(tab.at[eb.at[0]], rb,
                                      sem_g[b]).wait()

                @pl.when(g + 1 < CPW)
                def _():
                    pltpu.make_async_copy(me.at[0], ebn,
                                          sem_e[(q + 1) % 4]).wait()
                    pltpu.make_async_copy(mw.at[0], ewbn,
                                          sem_e[(q + 1) % 4]).wait()
                    @pl.when(g >= 1)
                    def _():
                        pltpu.make_async_copy(rbn, acc.at[ebn.at[1]],
                                              sem_s[1 - b]).wait()
                    pltpu.async_copy(tab.at[ebn.at[0]], rbn, sem_g[1 - b])

                _scale(ewbufs[q], rb)
                pltpu.async_copy(rb, acc.at[eb.at[1]], sem_s[b], add=True)

                @pl.when(g + 2 < CPW)
                def _():
                    pltpu.async_copy(me.at[g + 2], ebufs[(q + 2) % 4],
                                     sem_e[(q + 2) % 4])
                    pltpu.async_copy(mw.at[g + 2], ewbufs[(q + 2) % 4],
                                     sem_e[(q + 2) % 4])

        pltpu.make_async_copy(rows0, acc.at[ebuf0.at[1]], sem_s0).wait()
        pltpu.make_async_copy(rows1, acc.at[ebuf1.at[1]], sem_s1).wait()
        plsc.subcore_barrier()
        pltpu.sync_copy(acc.at[pl.ds(rbase, RPT)],
                        outp.at[pl.ds(rbase, RPT)])

    return pl.kernel(body, out_type=out_type, mesh=mesh,
                     scratch_types=scratch)


_sc_agg = _sc_agg_build()


def _chained_agg(tab, e4, ew4, token):
    if token is not None:
        tab, _ = lax.optimization_barrier((tab, token))
    (outp,) = _sc_agg(tab, e4, ew4)
    return outp


def _enc_body(x_ref, p_ref, cnt_ref, wt_ref, wb_ref, b_ref, o_ref):
    rinv = 1.0 / jnp.maximum(cnt_ref[:, 0:1], 1.0)
    agg = p_ref[...] * rinv
    o_ref[...] = (
        jnp.dot(x_ref[...], wt_ref[...], precision=_P)
        + jnp.dot(agg, wb_ref[...], precision=_P)
        + b_ref[...]
    )


def _encoder(xt, partial, counts, w, b):
    blk = lambda i: (i, 0)
    return pl.pallas_call(
        _enc_body,
        grid=(NBLK,),
        in_specs=[
            pl.BlockSpec((BLK, FF), blk),
            pl.BlockSpec((BLK, FF), blk),
            pl.BlockSpec((BLK, HH), blk),
            pl.BlockSpec((FF, HH), lambda i: (0, 0)),
            pl.BlockSpec((FF, HH), lambda i: (0, 0)),
            pl.BlockSpec((1, HH), lambda i: (0, 0)),
        ],
        out_specs=pl.BlockSpec((BLK, HH), blk),
        out_shape=jax.ShapeDtypeStruct((NP, HH), jnp.float32),
    )(xt, partial, counts, w[:FF], w[FF:], b.reshape(1, HH))


def _gate_body_t0(xs_ref, p_ref, cnt_ref, wxt_ref, wxb_ref, b_ref,
                  wo_ref, bo_ref, h_out, c_out, y_out):
    rinv = 1.0 / jnp.maximum(cnt_ref[:, 0:1], 1.0)
    aggx = p_ref[...] * rinv
    z = (
        jnp.dot(xs_ref[...], wxt_ref[...], precision=_P)
        + jnp.dot(aggx, wxb_ref[...], precision=_P)
        + b_ref[...]
    )
    i = jax.nn.sigmoid(z[:, 0 * HH:1 * HH])
    o = jax.nn.sigmoid(z[:, 2 * HH:3 * HH])
    g = jnp.tanh(z[:, 3 * HH:4 * HH])
    c = i * g
    h = o * jnp.tanh(c)
    c_out[...] = c
    h_out[...] = h
    y_out[...] = jnp.dot(h, wo_ref[...], precision=_P) + bo_ref[...]


def _gate_body(xs_ref, p_ref, h_ref, q_ref, cnt_ref, cst_ref,
               wxt_ref, wxb_ref, wht_ref, whb_ref, b_ref, wo_ref, bo_ref,
               h_out, c_out, y_out):
    rinv = 1.0 / jnp.maximum(cnt_ref[:, 0:1], 1.0)
    aggx = p_ref[...] * rinv
    aggh = q_ref[...] * rinv
    z = (
        jnp.dot(xs_ref[...], wxt_ref[...], precision=_P)
        + jnp.dot(aggx, wxb_ref[...], precision=_P)
        + jnp.dot(h_ref[...], wht_ref[...], precision=_P)
        + jnp.dot(aggh, whb_ref[...], precision=_P)
        + b_ref[...]
    )
    i = jax.nn.sigmoid(z[:, 0 * HH:1 * HH])
    f = jax.nn.sigmoid(z[:, 1 * HH:2 * HH])
    o = jax.nn.sigmoid(z[:, 2 * HH:3 * HH])
    g = jnp.tanh(z[:, 3 * HH:4 * HH])
    c = f * cst_ref[...] + i * g
    h = o * jnp.tanh(c)
    c_out[...] = c
    h_out[...] = h
    y_out[...] = jnp.dot(h, wo_ref[...], precision=_P) + bo_ref[...]


def _lstm_step(t, xs_t, aggxs_p, counts, h, aggh_p, cstate,
               wxt, wxb, wht, whb, bcat, wo, bo):
    blk = lambda i: (i, 0)
    fix = lambda i: (0, 0)
    wspec = pl.BlockSpec((HH, 4 * HH), fix)
    hb = pl.BlockSpec((BLK, HH), blk)
    out_specs = [
        pl.BlockSpec((BLK, HH), blk),
        pl.BlockSpec((BLK, HH), blk),
        pl.BlockSpec((BLK, 1), blk),
    ]
    out_shape = [
        jax.ShapeDtypeStruct((NP, HH), jnp.float32),
        jax.ShapeDtypeStruct((NP, HH), jnp.float32),
        jax.ShapeDtypeStruct((NP, 1), jnp.float32),
    ]
    if t == 0:
        return pl.pallas_call(
            _gate_body_t0,
            grid=(NBLK,),
            in_specs=[
                hb, hb, hb,
                wspec, wspec, pl.BlockSpec((1, 4 * HH), fix),
                pl.BlockSpec((HH, 1), fix), pl.BlockSpec((1, 1), fix),
            ],
            out_specs=out_specs,
            out_shape=out_shape,
        )(xs_t, aggxs_p, counts, wxt, wxb, bcat, wo, bo)
    return pl.pallas_call(
        _gate_body,
        grid=(NBLK,),
        in_specs=[
            hb, hb, hb, hb, hb, hb,
            wspec, wspec, wspec, wspec, pl.BlockSpec((1, 4 * HH), fix),
            pl.BlockSpec((HH, 1), fix), pl.BlockSpec((1, 1), fix),
        ],
        out_specs=out_specs,
        out_shape=out_shape,
    )(xs_t, aggxs_p, h, aggh_p, counts, cstate,
      wxt, wxb, wht, whb, bcat, wo, bo)


def kernel(x, edge_index, edge_attr, params):
    T = x.shape[0]
    npad = EP - EE
    src3 = jnp.pad(edge_index[0], (0, npad)).reshape(NSUB, CPW, CW)
    dst3 = jnp.pad(edge_index[1], (0, npad),
                   constant_values=NP - 1).reshape(NSUB, CPW, CW)
    ew4 = jnp.pad(edge_attr.reshape(EE), (0, npad)).reshape(NSUB, CPW, CW)
    e4 = jnp.stack([src3, dst3], axis=2)
    ew4ones = jnp.ones((NSUB, CPW, CW), jnp.float32)

    xp = jnp.pad(x, ((0, 0), (0, NP - NN), (0, 0)))

    def stack(keys):
        ws = [params[k] for k in keys]
        top = jnp.concatenate([w[:HH] for w in ws], axis=1)
        bot = jnp.concatenate([w[HH:] for w in ws], axis=1)
        return top, bot

    gk = ('i', 'f', 'o', 'g')
    wxt, wxb = stack([f'W_x{k}' for k in gk])
    wht, whb = stack([f'W_h{k}' for k in gk])
    bcat = jnp.concatenate(
        [params[f'b_x{k}'] + params[f'b_h{k}'] for k in gk]).reshape(1, 4 * HH)
    wo = params['W_out']
    bo = params['b_out'].reshape(1, 1)

    counts = _chained_agg(jnp.ones((NP, FF), jnp.float32), e4, ew4ones,
                          None)
    tok = counts

    xs = []
    for t in range(T):
        p = _chained_agg(xp[t], e4, ew4, tok)
        tok = p
        xs.append(_encoder(xp[t], p, counts,
                           params['W_sage'], params['b_sage']))

    aggxs = []
    for xs_t in xs:
        p = _chained_agg(xs_t, e4, ew4, tok)
        tok = p
        aggxs.append(p)

    h = None
    cstate = None
    hs, cs = [], []
    y = None
    for t in range(T):
        aggh_p = None
        if t > 0:
            aggh_p = _chained_agg(h, e4, ew4, tok)
            tok = aggh_p
        h, cstate, y = _lstm_step(t, xs[t], aggxs[t], counts, h, aggh_p,
                                  cstate, wxt, wxb, wht, whb, bcat, wo, bo)
        hs.append(h[:NN])
        cs.append(cstate[:NN])

    hseq = jnp.stack(hs, axis=0)
    cseq = jnp.stack(cs, axis=0)
    out = y[:NN].reshape(1, NN, 1)
    return (out, hseq, cseq)


# speedup vs baseline: 6.7348x; 1.0413x over previous
import functools

import jax
import jax.numpy as jnp
from jax import lax
from jax.experimental import pallas as pl
from jax.experimental.pallas import tpu as pltpu
from jax.experimental.pallas import tpu_sc as plsc

NN = 10000
NP = 10240
EE = 160000
FF = 128
HH = 128
BLK = 640
NBLK = NP // BLK

NSUB = 16
CW = 128
CPW = 80
EP = NSUB * CPW * CW
RPT = NP // NSUB

_P = jax.lax.Precision.HIGHEST


ZR = 64


def _sc_agg_build():
    mesh = plsc.VectorSubcoreMesh(core_axis_name="c", subcore_axis_name="s",
                                  num_cores=1)
    out_type = [jax.ShapeDtypeStruct((NP, HH), jnp.float32)]
    scratch = [
        pltpu.VMEM((2, CW), jnp.int32),
        pltpu.VMEM((2, CW), jnp.int32),
        pltpu.VMEM((2, CW), jnp.int32),
        pltpu.VMEM((2, CW), jnp.int32),
        pltpu.VMEM((CW,), jnp.float32),
        pltpu.VMEM((CW,), jnp.float32),
        pltpu.VMEM((CW,), jnp.float32),
        pltpu.VMEM((CW,), jnp.float32),
        pltpu.VMEM((CW, HH), jnp.float32),
        pltpu.VMEM((CW, HH), jnp.float32),
        pltpu.VMEM((ZR, HH), jnp.float32),
        pltpu.VMEM_SHARED((NP, HH), jnp.float32),
        pltpu.SemaphoreType.DMA,
        pltpu.SemaphoreType.DMA,
        pltpu.SemaphoreType.DMA,
        pltpu.SemaphoreType.DMA,
        pltpu.SemaphoreType.DMA,
        pltpu.SemaphoreType.DMA,
        pltpu.SemaphoreType.DMA,
        pltpu.SemaphoreType.DMA,
    ]

    def body(tab, e4, ew4, outp,
             ebuf0, ebuf1, ebuf2, ebuf3, ewbuf0, ewbuf1, ewbuf2, ewbuf3,
             rows0, rows1, zbuf, acc,
             sem_e0, sem_e1, sem_e2, sem_e3, sem_g0, sem_g1,
             sem_s0, sem_s1):
        s = lax.axis_index("s")
        rbase = pl.multiple_of(s * RPT, 8)
        ebufs = (ebuf0, ebuf1, ebuf2, ebuf3)
        ewbufs = (ewbuf0, ewbuf1, ewbuf2, ewbuf3)
        rows = (rows0, rows1)
        sem_e = (sem_e0, sem_e1, sem_e2, sem_e3)
        sem_g = (sem_g0, sem_g1)
        sem_s = (sem_s0, sem_s1)
        me = e4.at[s]
        mw = ew4.at[s]

        zero16 = jnp.zeros((16,), jnp.float32)

        def fill(i, cc):
            for kk in range(HH // 16):
                zbuf[i, pl.ds(kk * 16, 16)] = zero16
            return cc
        lax.fori_loop(0, ZR, fill, 0)

        @pl.loop(0, RPT // ZR)
        def zero(repi):
            zoff = pl.multiple_of(rbase + repi * ZR, 8)
            pltpu.sync_copy(zbuf, acc.at[pl.ds(zoff, ZR)])
        plsc.subcore_barrier()

        pltpu.sync_copy(me.at[0], ebuf0)
        pltpu.sync_copy(mw.at[0], ewbuf0)
        pltpu.async_copy(tab.at[ebuf0.at[0]], rows0, sem_g0)
        pltpu.async_copy(me.at[1], ebuf1, sem_e1)
        pltpu.async_copy(mw.at[1], ewbuf1, sem_e1)

        def _scale(ewb, rb):
            def scale(j, c2):
                ew16 = ewb[pl.ds(j * 16, 16)]
                for l in range(16):
                    ew_s = ew16[l]
                    e = j * 16 + l
                    for kk in range(HH // 16):
                        sl = pl.ds(kk * 16, 16)
                        rb[e, sl] = rb[e, sl] * ew_s
                return c2
            lax.fori_loop(0, CW // 16, scale, 0)

        @pl.loop(0, CPW, step=4)
        def iters(g0):
            for q in range(4):
                g = g0 + q
                b = q % 2
                eb, rb = ebufs[q], rows[b]
                ebn, rbn = ebufs[(q + 1) % 4], rows[1 - b]
                ewbn = ewbufs[(q + 1) % 4]
                @pl.when(g + 1 < CPW)
                def _():
                    pltpu.make_async_copy(me.at[0], ebn,
                                          sem_e[(q + 1) % 4]).wait()
                    pltpu.make_async_copy(mw.at[0], ewbn,
                                          sem_e[(q + 1) % 4]).wait()
                    @pl.when(g >= 1)
                    def _():
                        pltpu.make_async_copy(rbn, acc.at[ebn.at[1]],
                                              sem_s[1 - b]).wait()
                    pltpu.async_copy(tab.at[ebn.at[0]], rbn, sem_g[1 - b])

                pltpu.make_async_copy(tab.at[eb.at[0]], rb,
                                      sem_g[b]).wait()
                _scale(ewbufs[q], rb)
                pltpu.async_copy(rb, acc.at[eb.at[1]], sem_s[b], add=True)

                @pl.when(g + 2 < CPW)
                def _():
                    pltpu.async_copy(me.at[g + 2], ebufs[(q + 2) % 4],
                                     sem_e[(q + 2) % 4])
                    pltpu.async_copy(mw.at[g + 2], ewbufs[(q + 2) % 4],
                                     sem_e[(q + 2) % 4])

        pltpu.make_async_copy(rows0, acc.at[ebuf0.at[1]], sem_s0).wait()
        pltpu.make_async_copy(rows1, acc.at[ebuf1.at[1]], sem_s1).wait()
        plsc.subcore_barrier()
        pltpu.sync_copy(acc.at[pl.ds(rbase, RPT)],
                        outp.at[pl.ds(rbase, RPT)])

    return pl.kernel(body, out_type=out_type, mesh=mesh,
                     scratch_types=scratch)


_sc_agg = _sc_agg_build()


def _chained_agg(tab, e4, ew4, token):
    if token is not None:
        tab, _ = lax.optimization_barrier((tab, token))
    (outp,) = _sc_agg(tab, e4, ew4)
    return outp


def _enc_body(x_ref, p_ref, cnt_ref, wt_ref, wb_ref, b_ref, o_ref):
    rinv = 1.0 / jnp.maximum(cnt_ref[:, 0:1], 1.0)
    agg = p_ref[...] * rinv
    o_ref[...] = (
        jnp.dot(x_ref[...], wt_ref[...], precision=_P)
        + jnp.dot(agg, wb_ref[...], precision=_P)
        + b_ref[...]
    )


def _encoder(xt, partial, counts, w, b):
    blk = lambda i: (i, 0)
    return pl.pallas_call(
        _enc_body,
        grid=(NBLK,),
        in_specs=[
            pl.BlockSpec((BLK, FF), blk),
            pl.BlockSpec((BLK, FF), blk),
            pl.BlockSpec((BLK, HH), blk),
            pl.BlockSpec((FF, HH), lambda i: (0, 0)),
            pl.BlockSpec((FF, HH), lambda i: (0, 0)),
            pl.BlockSpec((1, HH), lambda i: (0, 0)),
        ],
        out_specs=pl.BlockSpec((BLK, HH), blk),
        out_shape=jax.ShapeDtypeStruct((NP, HH), jnp.float32),
    )(xt, partial, counts, w[:FF], w[FF:], b.reshape(1, HH))


def _gate_body_t0(xs_ref, p_ref, cnt_ref, wxt_ref, wxb_ref, b_ref,
                  wo_ref, bo_ref, h_out, c_out, y_out):
    rinv = 1.0 / jnp.maximum(cnt_ref[:, 0:1], 1.0)
    aggx = p_ref[...] * rinv
    z = (
        jnp.dot(xs_ref[...], wxt_ref[...], precision=_P)
        + jnp.dot(aggx, wxb_ref[...], precision=_P)
        + b_ref[...]
    )
    i = jax.nn.sigmoid(z[:, 0 * HH:1 * HH])
    o = jax.nn.sigmoid(z[:, 2 * HH:3 * HH])
    g = jnp.tanh(z[:, 3 * HH:4 * HH])
    c = i * g
    h = o * jnp.tanh(c)
    c_out[...] = c
    h_out[...] = h
    y_out[...] = jnp.dot(h, wo_ref[...], precision=_P) + bo_ref[...]


def _gate_body(xs_ref, p_ref, h_ref, q_ref, cnt_ref, cst_ref,
               wxt_ref, wxb_ref, wht_ref, whb_ref, b_ref, wo_ref, bo_ref,
               h_out, c_out, y_out):
    rinv = 1.0 / jnp.maximum(cnt_ref[:, 0:1], 1.0)
    aggx = p_ref[...] * rinv
    aggh = q_ref[...] * rinv
    z = (
        jnp.dot(xs_ref[...], wxt_ref[...], precision=_P)
        + jnp.dot(aggx, wxb_ref[...], precision=_P)
        + jnp.dot(h_ref[...], wht_ref[...], precision=_P)
        + jnp.dot(aggh, whb_ref[...], precision=_P)
        + b_ref[...]
    )
    i = jax.nn.sigmoid(z[:, 0 * HH:1 * HH])
    f = jax.nn.sigmoid(z[:, 1 * HH:2 * HH])
    o = jax.nn.sigmoid(z[:, 2 * HH:3 * HH])
    g = jnp.tanh(z[:, 3 * HH:4 * HH])
    c = f * cst_ref[...] + i * g
    h = o * jnp.tanh(c)
    c_out[...] = c
    h_out[...] = h
    y_out[...] = jnp.dot(h, wo_ref[...], precision=_P) + bo_ref[...]


def _lstm_step(t, xs_t, aggxs_p, counts, h, aggh_p, cstate,
               wxt, wxb, wht, whb, bcat, wo, bo):
    blk = lambda i: (i, 0)
    fix = lambda i: (0, 0)
    wspec = pl.BlockSpec((HH, 4 * HH), fix)
    hb = pl.BlockSpec((BLK, HH), blk)
    out_specs = [
        pl.BlockSpec((BLK, HH), blk),
        pl.BlockSpec((BLK, HH), blk),
        pl.BlockSpec((BLK, 1), blk),
    ]
    out_shape = [
        jax.ShapeDtypeStruct((NP, HH), jnp.float32),
        jax.ShapeDtypeStruct((NP, HH), jnp.float32),
        jax.ShapeDtypeStruct((NP, 1), jnp.float32),
    ]
    if t == 0:
        return pl.pallas_call(
            _gate_body_t0,
            grid=(NBLK,),
            in_specs=[
                hb, hb, hb,
                wspec, wspec, pl.BlockSpec((1, 4 * HH), fix),
                pl.BlockSpec((HH, 1), fix), pl.BlockSpec((1, 1), fix),
            ],
            out_specs=out_specs,
            out_shape=out_shape,
        )(xs_t, aggxs_p, counts, wxt, wxb, bcat, wo, bo)
    return pl.pallas_call(
        _gate_body,
        grid=(NBLK,),
        in_specs=[
            hb, hb, hb, hb, hb, hb,
            wspec, wspec, wspec, wspec, pl.BlockSpec((1, 4 * HH), fix),
            pl.BlockSpec((HH, 1), fix), pl.BlockSpec((1, 1), fix),
        ],
        out_specs=out_specs,
        out_shape=out_shape,
    )(xs_t, aggxs_p, h, aggh_p, counts, cstate,
      wxt, wxb, wht, whb, bcat, wo, bo)


def kernel(x, edge_index, edge_attr, params):
    T = x.shape[0]
    npad = EP - EE
    src3 = jnp.pad(edge_index[0], (0, npad)).reshape(NSUB, CPW, CW)
    dst3 = jnp.pad(edge_index[1], (0, npad),
                   constant_values=NP - 1).reshape(NSUB, CPW, CW)
    ew4 = jnp.pad(edge_attr.reshape(EE), (0, npad)).reshape(NSUB, CPW, CW)
    e4 = jnp.stack([src3, dst3], axis=2)
    ew4ones = jnp.ones((NSUB, CPW, CW), jnp.float32)

    xp = jnp.pad(x, ((0, 0), (0, NP - NN), (0, 0)))

    def stack(keys):
        ws = [params[k] for k in keys]
        top = jnp.concatenate([w[:HH] for w in ws], axis=1)
        bot = jnp.concatenate([w[HH:] for w in ws], axis=1)
        return top, bot

    gk = ('i', 'f', 'o', 'g')
    wxt, wxb = stack([f'W_x{k}' for k in gk])
    wht, whb = stack([f'W_h{k}' for k in gk])
    bcat = jnp.concatenate(
        [params[f'b_x{k}'] + params[f'b_h{k}'] for k in gk]).reshape(1, 4 * HH)
    wo = params['W_out']
    bo = params['b_out'].reshape(1, 1)

    counts = _chained_agg(jnp.ones((NP, FF), jnp.float32), e4, ew4ones,
                          None)
    tok = counts

    xs = []
    for t in range(T):
        p = _chained_agg(xp[t], e4, ew4, tok)
        tok = p
        xs.append(_encoder(xp[t], p, counts,
                           params['W_sage'], params['b_sage']))

    aggxs = []
    for xs_t in xs:
        p = _chained_agg(xs_t, e4, ew4, tok)
        tok = p
        aggxs.append(p)

    h = None
    cstate = None
    hs, cs = [], []
    y = None
    for t in range(T):
        aggh_p = None
        if t > 0:
            aggh_p = _chained_agg(h, e4, ew4, tok)
            tok = aggh_p
        h, cstate, y = _lstm_step(t, xs[t], aggxs[t], counts, h, aggh_p,
                                  cstate, wxt, wxb, wht, whb, bcat, wo, bo)
        hs.append(h[:NN])
        cs.append(cstate[:NN])

    hseq = jnp.stack(hs, axis=0)
    cseq = jnp.stack(cs, axis=0)
    out = y[:NN].reshape(1, NN, 1)
    return (out, hseq, cseq)
